# baseline (speedup 1.0000x reference)
import functools

import jax
import jax.numpy as jnp
from jax import lax
from jax.experimental import pallas as pl
from jax.experimental.pallas import tpu as pltpu
from jax.experimental.pallas import tpu_sc as plsc

HIDDEN_DIM = 768
N_EXPERTS = 64
TOPK = 8
TOKEN_BLOCK = 4096
N_TOKENS = 32768

_info = plsc.get_sparse_core_info()
_NC, _NS, _L = _info.num_cores, _info.num_subcores, _info.num_lanes
_NW = _NC * _NS
_CHUNK = N_TOKENS // _NW
_GROUPS = _CHUNK // 16


def _logits_block(x_ref, w_ref, out_ref):
    out_ref[...] = jax.lax.dot_general(
        w_ref[...], x_ref[...], (((1,), (1,)), ((), ())),
        preferred_element_type=jnp.float32,
    )


def _tc_logits(x_flat, W_router):
    grid = (N_TOKENS // TOKEN_BLOCK,)
    return pl.pallas_call(
        _logits_block,
        grid=grid,
        in_specs=[
            pl.BlockSpec((TOKEN_BLOCK, HIDDEN_DIM), lambda i: (i, 0)),
            pl.BlockSpec((N_EXPERTS, HIDDEN_DIM), lambda i: (0, 0)),
        ],
        out_specs=pl.BlockSpec((N_EXPERTS, TOKEN_BLOCK), lambda i: (0, i)),
        out_shape=jax.ShapeDtypeStruct((N_EXPERTS, N_TOKENS), jnp.float32),
        compiler_params=pltpu.CompilerParams(
            dimension_semantics=("arbitrary",),
        ),
    )(x_flat, W_router)


_mesh = plsc.VectorSubcoreMesh(core_axis_name="c", subcore_axis_name="s")


@functools.partial(
    pl.kernel,
    mesh=_mesh,
    out_type=[
        jax.ShapeDtypeStruct((TOPK, N_TOKENS), jnp.float32),
        jax.ShapeDtypeStruct((TOPK, N_TOKENS), jnp.int32),
    ],
    scratch_types=[
        pltpu.VMEM((N_EXPERTS * _CHUNK,), jnp.float32),
        pltpu.VMEM((_CHUNK * TOPK,), jnp.float32),
        pltpu.VMEM((_CHUNK * TOPK,), jnp.int32),
        pltpu.SemaphoreType.DMA,
    ],
)
def _sc_route(logits_hbm, wts_hbm, idx_hbm, buf, wbuf, ibuf, sem):
    wid = lax.axis_index("s") * _NC + lax.axis_index("c")
    base = wid * _CHUNK

    copies = [
        pltpu.make_async_copy(
            logits_hbm.at[e, pl.ds(base, _CHUNK)],
            buf.at[pl.ds(e * _CHUNK, _CHUNK)],
            sem,
        )
        for e in range(N_EXPERTS)
    ]
    for c in copies:
        c.start()
    for c in copies:
        c.wait()

    lane = lax.broadcasted_iota(jnp.int32, (_L,), 0)
    neg_inf = jnp.full((_L,), -jnp.inf, dtype=jnp.float32)
    zero_i = jnp.zeros((_L,), dtype=jnp.int32)

    def group_body(g, carry):
        off = g * _L

        def expert_body(e, st):
            vals, idxs = st
            v = buf[pl.ds(e * _CHUNK + off, _L)]
            ei = jnp.full((_L,), 0, dtype=jnp.int32) + e
            for j in range(TOPK):
                m = v > vals[j]
                nv = jnp.maximum(vals[j], v)
                v = jnp.minimum(vals[j], v)
                ni = jnp.where(m, ei, idxs[j])
                ei = jnp.where(m, idxs[j], ei)
                vals = vals[:j] + (nv,) + vals[j + 1 :]
                idxs = idxs[:j] + (ni,) + idxs[j + 1 :]
            return vals, idxs

        init = ((neg_inf,) * TOPK, (zero_i,) * TOPK)
        vals, idxs = lax.fori_loop(0, N_EXPERTS, expert_body, init)

        es = [jnp.full((_L,), 1.0, dtype=jnp.float32)]
        es += [jnp.exp(vals[j] - vals[0]) for j in range(1, TOPK)]
        s = es[0]
        for e in es[1:]:
            s = s + e
        r = jnp.full((_L,), 1.0, dtype=jnp.float32) / s

        for j in range(TOPK):
            wbuf[pl.ds(j * _CHUNK + off, _L)] = es[j] * r
            ibuf[pl.ds(j * _CHUNK + off, _L)] = idxs[j]
        return carry

    lax.fori_loop(0, _GROUPS, group_body, 0)

    for j in range(TOPK):
        pltpu.sync_copy(
            wbuf.at[pl.ds(j * _CHUNK, _CHUNK)],
            wts_hbm.at[j, pl.ds(base, _CHUNK)],
        )
        pltpu.sync_copy(
            ibuf.at[pl.ds(j * _CHUNK, _CHUNK)],
            idx_hbm.at[j, pl.ds(base, _CHUNK)],
        )


def kernel(x, W_router):
    x_flat = x.reshape(N_TOKENS, HIDDEN_DIM)
    logits_t = _tc_logits(x_flat, W_router)
    wts_t, idx_t = _sc_route(logits_t)
    return wts_t.T, idx_t.T


# speedup vs baseline: 1.0134x; 1.0134x over previous
import functools

import jax
import jax.numpy as jnp
from jax import lax
from jax.experimental import pallas as pl
from jax.experimental.pallas import tpu as pltpu
from jax.experimental.pallas import tpu_sc as plsc

HIDDEN_DIM = 768
N_EXPERTS = 64
TOPK = 8
TOKEN_BLOCK = 4096
N_TOKENS = 32768
N_CHUNKS = 2
CTOK = N_TOKENS // N_CHUNKS

_info = plsc.get_sparse_core_info()
_NC, _NS, _L = _info.num_cores, _info.num_subcores, _info.num_lanes
_NW = _NC * _NS
_CHUNK = CTOK // _NW
_GROUPS = _CHUNK // _L

_mesh = plsc.VectorSubcoreMesh(core_axis_name="c", subcore_axis_name="s")


def _logits_block(x_ref, w_ref, out_ref):
    out_ref[...] = jax.lax.dot_general(
        w_ref[...], x_ref[...], (((1,), (1,)), ((), ())),
        preferred_element_type=jnp.float32,
    )


def _tc_logits_chunk(x_flat, W_router, chunk):
    grid = (CTOK // TOKEN_BLOCK,)
    base_blk = chunk * (CTOK // TOKEN_BLOCK)
    return pl.pallas_call(
        _logits_block,
        grid=grid,
        in_specs=[
            pl.BlockSpec(
                (TOKEN_BLOCK, HIDDEN_DIM), lambda i: (base_blk + i, 0)
            ),
            pl.BlockSpec((N_EXPERTS, HIDDEN_DIM), lambda i: (0, 0)),
        ],
        out_specs=pl.BlockSpec((N_EXPERTS, TOKEN_BLOCK), lambda i: (0, i)),
        out_shape=jax.ShapeDtypeStruct((N_EXPERTS, CTOK), jnp.float32),
        compiler_params=pltpu.CompilerParams(
            dimension_semantics=("arbitrary",),
        ),
    )(x_flat, W_router)


@functools.partial(
    pl.kernel,
    mesh=_mesh,
    out_type=[
        jax.ShapeDtypeStruct((TOPK, CTOK), jnp.float32),
        jax.ShapeDtypeStruct((TOPK, CTOK), jnp.int32),
    ],
    scratch_types=[
        pltpu.VMEM((N_EXPERTS * _CHUNK,), jnp.float32),
        pltpu.VMEM((_CHUNK * TOPK,), jnp.float32),
        pltpu.VMEM((_CHUNK * TOPK,), jnp.int32),
        pltpu.SemaphoreType.DMA,
    ],
)
def _sc_route(logits_hbm, wts_hbm, idx_hbm, buf, wbuf, ibuf, sem):
    wid = lax.axis_index("s") * _NC + lax.axis_index("c")
    base = wid * _CHUNK

    copies = [
        pltpu.make_async_copy(
            logits_hbm.at[e, pl.ds(base, _CHUNK)],
            buf.at[pl.ds(e * _CHUNK, _CHUNK)],
            sem,
        )
        for e in range(N_EXPERTS)
    ]
    for c in copies:
        c.start()
    for c in copies:
        c.wait()

    neg_inf = jnp.full((_L,), -jnp.inf, dtype=jnp.float32)
    zero_i = jnp.zeros((_L,), dtype=jnp.int32)

    def group_body(g, carry):
        off = g * _L

        def expert_body(e, st):
            vals, idxs = st
            v = buf[pl.ds(e * _CHUNK + off, _L)]
            ei = jnp.full((_L,), 0, dtype=jnp.int32) + e
            for j in range(TOPK):
                m = v > vals[j]
                nv = jnp.maximum(vals[j], v)
                v = jnp.minimum(vals[j], v)
                ni = jnp.where(m, ei, idxs[j])
                ei = jnp.where(m, idxs[j], ei)
                vals = vals[:j] + (nv,) + vals[j + 1 :]
                idxs = idxs[:j] + (ni,) + idxs[j + 1 :]
            return vals, idxs

        init = ((neg_inf,) * TOPK, (zero_i,) * TOPK)
        vals, idxs = lax.fori_loop(0, N_EXPERTS, expert_body, init)

        es = [jnp.full((_L,), 1.0, dtype=jnp.float32)]
        es += [jnp.exp(vals[j] - vals[0]) for j in range(1, TOPK)]
        s = es[0]
        for e in es[1:]:
            s = s + e
        r = jnp.full((_L,), 1.0, dtype=jnp.float32) / s

        for j in range(TOPK):
            wbuf[pl.ds(j * _CHUNK + off, _L)] = es[j] * r
            ibuf[pl.ds(j * _CHUNK + off, _L)] = idxs[j]
        return carry

    lax.fori_loop(0, _GROUPS, group_body, 0)

    for j in range(TOPK):
        pltpu.sync_copy(
            wbuf.at[pl.ds(j * _CHUNK, _CHUNK)],
            wts_hbm.at[j, pl.ds(base, _CHUNK)],
        )
        pltpu.sync_copy(
            ibuf.at[pl.ds(j * _CHUNK, _CHUNK)],
            idx_hbm.at[j, pl.ds(base, _CHUNK)],
        )


def kernel(x, W_router):
    x_flat = x.reshape(N_TOKENS, HIDDEN_DIM)
    wts_parts = []
    idx_parts = []
    logits = [_tc_logits_chunk(x_flat, W_router, c) for c in range(N_CHUNKS)]
    for c in range(N_CHUNKS):
        w_t, i_t = _sc_route(logits[c])
        wts_parts.append(w_t.T)
        idx_parts.append(i_t.T)
    return (
        jnp.concatenate(wts_parts, axis=0),
        jnp.concatenate(idx_parts, axis=0),
    )


# speedup vs baseline: 1.3225x; 1.3051x over previous
import jax
import jax.numpy as jnp
from jax.experimental import pallas as pl
from jax.experimental.pallas import tpu as pltpu

HIDDEN_DIM = 768
N_EXPERTS = 64
TOPK = 8
TOKEN_BLOCK = 4096


def _router_block(x_ref, w_ref, wts_ref, idx_ref):
    xb = x_ref[...]
    wb = w_ref[...]
    logits = jax.lax.dot_general(
        wb, xb, (((1,), (1,)), ((), ())), preferred_element_type=jnp.float32
    )
    t = logits.shape[1]
    fiota = jax.lax.broadcasted_iota(jnp.int32, (N_EXPERTS, t), 0).astype(
        jnp.float32
    )
    neg_inf = jnp.float32(-jnp.inf)

    cur = logits
    vals = []
    idxs = []
    for k in range(TOPK):
        m = jnp.max(cur, axis=0, keepdims=True)
        i = jnp.min(
            jnp.where(cur == m, fiota, jnp.float32(N_EXPERTS)),
            axis=0,
            keepdims=True,
        )
        vals.append(m)
        idxs.append(i)
        if k + 1 < TOPK:
            cur = jnp.where(fiota == i, neg_inf, cur)

    es = [jnp.ones_like(vals[0])]
    es += [jnp.exp(v - vals[0]) for v in vals[1:]]
    s = es[0]
    for e in es[1:]:
        s = s + e
    r = jnp.float32(1.0) / s
    wts_t = jnp.concatenate([e * r for e in es], axis=0)
    idx_t = jnp.concatenate(idxs, axis=0)
    wts_ref[...] = wts_t.T
    idx_ref[...] = idx_t.T.astype(jnp.int32)


def kernel(x, W_router):
    n_tokens = x.shape[0] * x.shape[1]
    x_flat = x.reshape(n_tokens, HIDDEN_DIM)
    grid = (n_tokens // TOKEN_BLOCK,)
    wts_t, idx_t = pl.pallas_call(
        _router_block,
        grid=grid,
        in_specs=[
            pl.BlockSpec((TOKEN_BLOCK, HIDDEN_DIM), lambda i: (i, 0)),
            pl.BlockSpec((N_EXPERTS, HIDDEN_DIM), lambda i: (0, 0)),
        ],
        out_specs=[
            pl.BlockSpec((TOKEN_BLOCK, TOPK), lambda i: (i, 0)),
            pl.BlockSpec((TOKEN_BLOCK, TOPK), lambda i: (i, 0)),
        ],
        out_shape=[
            jax.ShapeDtypeStruct((n_tokens, TOPK), jnp.float32),
            jax.ShapeDtypeStruct((n_tokens, TOPK), jnp.int32),
        ],
        compiler_params=pltpu.CompilerParams(
            dimension_semantics=("arbitrary",),
        ),
    )(x_flat, W_router)
    return wts_t, idx_t
